# speedup vs baseline: 1.1673x; 1.1673x over previous
import functools

import jax
import jax.numpy as jnp
from jax import lax
from jax.experimental import pallas as pl
from jax.experimental.pallas import tpu as pltpu
from jax.experimental.pallas import tpu_sc as plsc

_BATCH = 16384
_NUM_CORES = 2
_NUM_SUBCORES = 16
_NUM_WORKERS = _NUM_CORES * _NUM_SUBCORES
_PER_WORKER = _BATCH // _NUM_WORKERS
_CHUNK = 128
_NUM_CHUNKS = _PER_WORKER // _CHUNK

_mesh = plsc.VectorSubcoreMesh(core_axis_name="c", subcore_axis_name="s")


@functools.partial(
    pl.kernel,
    mesh=_mesh,
    out_type=jax.ShapeDtypeStruct((_NUM_WORKERS, _NUM_CHUNKS, _CHUNK), jnp.float32),
    scratch_types=[
        pltpu.VMEM((_NUM_CHUNKS, _CHUNK), jnp.int32),
        pltpu.VMEM((_NUM_CHUNKS, _CHUNK), jnp.float32),
        pltpu.SemaphoreType.DMA,
    ],
)
def _gather_kernel(w_hbm, idx_hbm, out_hbm, idx_v, vals_v, sem):
    wid = lax.axis_index("s") * _NUM_CORES + lax.axis_index("c")
    pltpu.sync_copy(idx_hbm.at[wid], idx_v)
    copies = [
        pltpu.async_copy(w_hbm.at[idx_v.at[j]], vals_v.at[j], sem)
        for j in range(_NUM_CHUNKS)
    ]
    for c in copies:
        c.wait()
    pltpu.sync_copy(vals_v, out_hbm.at[wid])


def kernel(cur_step, W):
    idx = cur_step.astype(jnp.int32).reshape(_NUM_WORKERS, _NUM_CHUNKS, _CHUNK)
    table = W.reshape(-1)
    out = _gather_kernel(table, idx)
    return out.reshape(_BATCH, 1)
